# baseline (speedup 1.0000x reference)
import functools

import jax
import jax.numpy as jnp
from jax import lax
from jax.experimental import pallas as pl
from jax.experimental.pallas import tpu as pltpu
from jax.experimental.pallas import tpu_sc as plsc

C = 28
N = 131072
B = 2048
HI = 8.0
SCALE = B / HI
L = 16
NTILES = 32
CNT_TOT = 4112
CHUNK = 32768
NCHUNKS = N // CHUNK
U = 4

_mesh = plsc.VectorSubcoreMesh(core_axis_name="c", subcore_axis_name="s")


@functools.partial(
    pl.kernel,
    out_type=jax.ShapeDtypeStruct((NTILES, L), jnp.float32),
    mesh=_mesh,
    scratch_types=[
        pltpu.VMEM((CHUNK,), jnp.bfloat16),
        pltpu.VMEM((CHUNK,), jnp.bfloat16),
        pltpu.VMEM((CHUNK,), jnp.bfloat16),
        pltpu.VMEM((CHUNK,), jnp.bfloat16),
        pltpu.VMEM((CNT_TOT,), jnp.float32),
        pltpu.VMEM((L,), jnp.float32),
        pltpu.SemaphoreType.DMA,
        pltpu.SemaphoreType.DMA,
    ],
    compiler_params=pltpu.CompilerParams(needs_layout_passes=False),
)
def _lovasz_sc(logits_t, labels_t, out, lbuf0, lbuf1, ybuf0, ybuf1, cnt,
               ostage, sem0, sem1):
    wid = lax.axis_index("s") * 2 + lax.axis_index("c")
    row = jnp.minimum(wid, C - 1)
    active = (wid < C).astype(jnp.float32)

    zeros = jnp.zeros((L,), jnp.float32)
    ones = jnp.ones((L,), jnp.float32)
    iota = lax.iota(jnp.int32, L)
    sems = (sem0, sem1)

    @plsc.parallel_loop(0, CNT_TOT // L, unroll=8)
    def _zero_cnt(i):
        cnt[pl.ds(i * L, L)] = zeros

    lbufs = (lbuf0, lbuf1)
    ybufs = (ybuf0, ybuf1)

    def _start(ch):
        p = ch & 1
        off = row * N + ch * CHUNK
        hl = pltpu.async_copy(logits_t.at[pl.ds(off, CHUNK)], lbufs[p], sems[p])
        hy = pltpu.async_copy(labels_t.at[pl.ds(off, CHUNK)], ybufs[p], sems[p])
        return hl, hy

    def _hist_pair(p, i):
        lg2 = lbufs[p][pl.ds(i * 2 * L, 2 * L)]
        yy2 = ybufs[p][pl.ds(i * 2 * L, 2 * L)]
        lgs = plsc.unpack(lg2, format=plsc.PackFormat.INTERLEAVED)
        yfs = plsc.unpack(yy2, format=plsc.PackFormat.INTERLEAVED)
        for lg, yf in zip(lgs, yfs):
            yi = yf.astype(jnp.int32)
            t = lg * yf
            e = (1.0 + lg) - 2.0 * t
            s = e * SCALE + 1.0
            s = jnp.minimum(s, float(B))
            s = jnp.maximum(s, 0.0)
            v = s.astype(jnp.int32)
            idx = (2 * B) - (v + v) + yi
            plsc.addupdate_scatter(cnt, [idx], ones)

    pend = _start(0)
    for ch in range(NCHUNKS):
        nxt = _start(ch + 1) if ch + 1 < NCHUNKS else None
        pend[0].wait()
        pend[1].wait()
        plsc.parallel_loop(0, CHUNK // (2 * L), unroll=U)(
            functools.partial(_hist_pair, ch & 1))
        pend = nxt

    oddmask = (iota & 1).astype(jnp.float32)

    def _gsum(i, acc):
        return acc + cnt[pl.ds(i * L, L)] * oddmask

    g = jnp.sum(lax.fori_loop(0, CNT_TOT // L, _gsum, zeros))
    g2 = float(N) - g

    def _final(b, carry):
        pcar, ncar, acc = carry
        base = 2 * (b * L) + 2 * iota
        cn = plsc.load_gather(cnt, [base])
        cp = plsc.load_gather(cnt, [base + 1])
        pi = plsc.cumsum(cp) + pcar
        ni = plsc.cumsum(cn) + ncar
        pe = pi - cp
        ne = ni - cn
        den1 = jnp.maximum((g + ne) * (g + ne + cn), 1.0)
        mass1 = ((g - pe) * cn + cp * (g + ne)) / den1
        den2 = jnp.maximum((g2 + pe) * (g2 + pe + cp), 1.0)
        mass2 = ((g2 - ne) * cp + cn * (g2 + pe)) / den2
        bg = (b * L + iota).astype(jnp.float32)
        center = (B - 0.5) / SCALE - bg * (1.0 / SCALE)
        acc = acc + center * (mass1 + mass2)
        return pcar + jnp.sum(cp), ncar + jnp.sum(cn), acc

    _, _, acc = lax.fori_loop(
        0, B // L, _final, (jnp.float32(0.0), jnp.float32(0.0), zeros)
    )

    partial = jnp.sum(acc) * active * (1.0 / (2.0 * C))
    ostage[...] = jnp.where(iota == 0, partial, 0.0)
    pltpu.sync_copy(ostage, out.at[wid])


def kernel(logits, labels):
    logits_t = logits.astype(jnp.bfloat16).T.reshape(-1)
    labels_t = labels.astype(jnp.bfloat16).T.reshape(-1)
    parts = _lovasz_sc(logits_t, labels_t)
    return jnp.sum(parts)


# speedup vs baseline: 1.5217x; 1.5217x over previous
import functools

import jax
import jax.numpy as jnp
from jax import lax
from jax.experimental import pallas as pl
from jax.experimental.pallas import tpu as pltpu
from jax.experimental.pallas import tpu_sc as plsc

C = 28
N = 131072
B = 2048
HI = 8.0
SCALE = B / HI
L = 16
NTILES = 32
CNT_TOT = 4112
CHUNK = 8192
NCHUNKS = N // CHUNK
U = 8

_mesh = plsc.VectorSubcoreMesh(core_axis_name="c", subcore_axis_name="s")


@functools.partial(
    pl.kernel,
    out_type=jax.ShapeDtypeStruct((NTILES, L), jnp.float32),
    mesh=_mesh,
    scratch_types=[
        pltpu.VMEM((2, CHUNK), jnp.float32),
        pltpu.VMEM((2, CHUNK), jnp.int32),
        pltpu.VMEM((CNT_TOT,), jnp.float32),
        pltpu.VMEM((L,), jnp.float32),
        pltpu.SemaphoreType.DMA,
        pltpu.SemaphoreType.DMA,
    ],
    compiler_params=pltpu.CompilerParams(needs_layout_passes=False),
)
def _lovasz_sc(logits_t, labels_t, out, lbuf, ybuf, cnt, ostage, sem0, sem1):
    wid = lax.axis_index("s") * 2 + lax.axis_index("c")
    row = jnp.minimum(wid, C - 1)
    active = (wid < C).astype(jnp.float32)

    zeros = jnp.zeros((L,), jnp.float32)
    ones = jnp.ones((L,), jnp.float32)
    iota = lax.iota(jnp.int32, L)
    sems = (sem0, sem1)

    @plsc.parallel_loop(0, CNT_TOT // L, unroll=8)
    def _zero_cnt(i):
        cnt[pl.ds(i * L, L)] = zeros

    def _start(ch):
        p = ch & 1
        hl = pltpu.async_copy(
            logits_t.at[row, pl.ds(ch * CHUNK, CHUNK)], lbuf.at[p], sems[p])
        hy = pltpu.async_copy(
            labels_t.at[row, pl.ds(ch * CHUNK, CHUNK)], ybuf.at[p], sems[p])
        return hl, hy

    def _hist_group(p, i):
        lg = lbuf[p, pl.ds(i * L, L)]
        yi = ybuf[p, pl.ds(i * L, L)]
        yf = yi.astype(jnp.float32)
        t = lg * yf
        e = (1.0 + lg) - 2.0 * t
        s = e * SCALE + 1.0
        s = jnp.minimum(s, float(B))
        s = jnp.maximum(s, 0.0)
        v = s.astype(jnp.int32)
        idx = (2 * B) - (v + v) + yi
        plsc.addupdate_scatter(cnt, [idx], ones)

    pend = _start(0)
    for ch in range(NCHUNKS):
        nxt = _start(ch + 1) if ch + 1 < NCHUNKS else None
        pend[0].wait()
        pend[1].wait()
        plsc.parallel_loop(0, CHUNK // L, unroll=U)(
            functools.partial(_hist_group, ch & 1))
        pend = nxt

    oddmask = (iota & 1).astype(jnp.float32)

    def _gsum(i, acc):
        return acc + cnt[pl.ds(i * L, L)] * oddmask

    g = jnp.sum(lax.fori_loop(0, CNT_TOT // L, _gsum, zeros))
    g2 = float(N) - g

    def _final(b, carry):
        pcar, ncar, acc = carry
        base = 2 * (b * L) + 2 * iota
        cn = plsc.load_gather(cnt, [base])
        cp = plsc.load_gather(cnt, [base + 1])
        pi = plsc.cumsum(cp) + pcar
        ni = plsc.cumsum(cn) + ncar
        pe = pi - cp
        ne = ni - cn
        den1 = jnp.maximum((g + ne) * (g + ne + cn), 1.0)
        mass1 = ((g - pe) * cn + cp * (g + ne)) / den1
        den2 = jnp.maximum((g2 + pe) * (g2 + pe + cp), 1.0)
        mass2 = ((g2 - ne) * cp + cn * (g2 + pe)) / den2
        bg = (b * L + iota).astype(jnp.float32)
        center = (B - 0.5) / SCALE - bg * (1.0 / SCALE)
        acc = acc + center * (mass1 + mass2)
        return pcar + jnp.sum(cp), ncar + jnp.sum(cn), acc

    _, _, acc = lax.fori_loop(
        0, B // L, _final, (jnp.float32(0.0), jnp.float32(0.0), zeros)
    )

    partial = jnp.sum(acc) * active * (1.0 / (2.0 * C))
    ostage[...] = jnp.where(iota == 0, partial, 0.0)
    pltpu.sync_copy(ostage, out.at[wid])


def kernel(logits, labels):
    logits_t = logits.T
    labels_t = labels.astype(jnp.int32).T
    parts = _lovasz_sc(logits_t, labels_t)
    return jnp.sum(parts)


# speedup vs baseline: 1.6300x; 1.0712x over previous
import functools

import jax
import jax.numpy as jnp
from jax import lax
from jax.experimental import pallas as pl
from jax.experimental.pallas import tpu as pltpu
from jax.experimental.pallas import tpu_sc as plsc

C = 28
N = 131072
B = 1024
HI = 8.0
SCALE = B / HI
L = 16
NTILES = 32
CNT_TOT = 2064
CHUNK = 16384
NCHUNKS = N // CHUNK
U = 8

_mesh = plsc.VectorSubcoreMesh(core_axis_name="c", subcore_axis_name="s")


@functools.partial(
    pl.kernel,
    out_type=jax.ShapeDtypeStruct((NTILES, L), jnp.float32),
    mesh=_mesh,
    scratch_types=[
        pltpu.VMEM((2, CHUNK), jnp.float32),
        pltpu.VMEM((2, CHUNK), jnp.int32),
        pltpu.VMEM((CNT_TOT,), jnp.float32),
        pltpu.VMEM((L,), jnp.float32),
        pltpu.SemaphoreType.DMA,
        pltpu.SemaphoreType.DMA,
    ],
    compiler_params=pltpu.CompilerParams(needs_layout_passes=False),
)
def _lovasz_sc(logits_t, labels_t, out, lbuf, ybuf, cnt, ostage, sem0, sem1):
    wid = lax.axis_index("s") * 2 + lax.axis_index("c")
    row = jnp.minimum(wid, C - 1)
    active = (wid < C).astype(jnp.float32)

    zeros = jnp.zeros((L,), jnp.float32)
    ones = jnp.ones((L,), jnp.float32)
    iota = lax.iota(jnp.int32, L)
    sems = (sem0, sem1)

    @plsc.parallel_loop(0, CNT_TOT // L, unroll=8)
    def _zero_cnt(i):
        cnt[pl.ds(i * L, L)] = zeros

    def _start(ch):
        p = ch & 1
        hl = pltpu.async_copy(
            logits_t.at[row, pl.ds(ch * CHUNK, CHUNK)], lbuf.at[p], sems[p])
        hy = pltpu.async_copy(
            labels_t.at[row, pl.ds(ch * CHUNK, CHUNK)], ybuf.at[p], sems[p])
        return hl, hy

    def _hist_group(p, i):
        lg = lbuf[p, pl.ds(i * L, L)]
        yi = ybuf[p, pl.ds(i * L, L)]
        yf = yi.astype(jnp.float32)
        t = lg * yf
        e = (1.0 + lg) - 2.0 * t
        s = e * SCALE + 1.0
        s = jnp.minimum(s, float(B))
        s = jnp.maximum(s, 0.0)
        v = s.astype(jnp.int32)
        idx = (2 * B) - (v + v) + yi
        plsc.addupdate_scatter(cnt, [idx], ones)

    pend = _start(0)
    for ch in range(NCHUNKS):
        nxt = _start(ch + 1) if ch + 1 < NCHUNKS else None
        pend[0].wait()
        pend[1].wait()
        plsc.parallel_loop(0, CHUNK // L, unroll=U)(
            functools.partial(_hist_group, ch & 1))
        pend = nxt

    oddmask = (iota & 1).astype(jnp.float32)

    def _gsum(i, acc):
        return acc + cnt[pl.ds(i * L, L)] * oddmask

    g = jnp.sum(lax.fori_loop(0, CNT_TOT // L, _gsum, zeros))
    g2 = float(N) - g

    def _final(b, carry):
        pcar, ncar, acc = carry
        base = 2 * (b * L) + 2 * iota
        cn = plsc.load_gather(cnt, [base])
        cp = plsc.load_gather(cnt, [base + 1])
        pi = plsc.cumsum(cp) + pcar
        ni = plsc.cumsum(cn) + ncar
        pe = pi - cp
        ne = ni - cn
        den1 = jnp.maximum((g + ne) * (g + ne + cn), 1.0)
        mass1 = ((g - pe) * cn + cp * (g + ne)) / den1
        den2 = jnp.maximum((g2 + pe) * (g2 + pe + cp), 1.0)
        mass2 = ((g2 - ne) * cp + cn * (g2 + pe)) / den2
        bg = (b * L + iota).astype(jnp.float32)
        center = (B - 0.5) / SCALE - bg * (1.0 / SCALE)
        acc = acc + center * (mass1 + mass2)
        return pcar + jnp.sum(cp), ncar + jnp.sum(cn), acc

    _, _, acc = lax.fori_loop(
        0, B // L, _final, (jnp.float32(0.0), jnp.float32(0.0), zeros)
    )

    partial = jnp.sum(acc) * active * (1.0 / (2.0 * C))
    ostage[...] = jnp.where(iota == 0, partial, 0.0)
    pltpu.sync_copy(ostage, out.at[wid])


def kernel(logits, labels):
    logits_t = logits.T
    labels_t = labels.astype(jnp.int32).T
    parts = _lovasz_sc(logits_t, labels_t)
    return jnp.sum(parts)


# speedup vs baseline: 1.6354x; 1.0033x over previous
import functools

import jax
import jax.numpy as jnp
from jax import lax
from jax.experimental import pallas as pl
from jax.experimental.pallas import tpu as pltpu
from jax.experimental.pallas import tpu_sc as plsc

C = 28
N = 131072
B = 1024
HI = 8.0
SCALE = B / HI
L = 16
NTILES = 32
CNT_TOT = 2064
CHUNK = 16384
NCHUNKS = N // CHUNK
U = 8

_mesh = plsc.VectorSubcoreMesh(core_axis_name="c", subcore_axis_name="s")


@functools.partial(
    pl.kernel,
    out_type=jax.ShapeDtypeStruct((NTILES, L), jnp.float32),
    mesh=_mesh,
    scratch_types=[
        pltpu.VMEM((2, CHUNK), jnp.float32),
        pltpu.VMEM((2, CHUNK), jnp.int32),
        pltpu.VMEM((CNT_TOT,), jnp.float32),
        pltpu.VMEM((L,), jnp.float32),
        pltpu.SemaphoreType.DMA,
        pltpu.SemaphoreType.DMA,
    ],
    compiler_params=pltpu.CompilerParams(needs_layout_passes=False),
)
def _lovasz_sc(logits_t, labels_t, out, lbuf, ybuf, cnt, ostage, sem0, sem1):
    wid = lax.axis_index("s") * 2 + lax.axis_index("c")
    row = jnp.minimum(wid, C - 1)
    active = (wid < C).astype(jnp.float32)

    zeros = jnp.zeros((L,), jnp.float32)
    ones = jnp.ones((L,), jnp.float32)
    iota = lax.iota(jnp.int32, L)
    sems = (sem0, sem1)

    @plsc.parallel_loop(0, CNT_TOT // L, unroll=8)
    def _zero_cnt(i):
        cnt[pl.ds(i * L, L)] = zeros

    def _start(ch):
        p = ch & 1
        hl = pltpu.async_copy(
            logits_t.at[row, pl.ds(ch * CHUNK, CHUNK)], lbuf.at[p], sems[p])
        hy = pltpu.async_copy(
            labels_t.at[row, pl.ds(ch * CHUNK, CHUNK)], ybuf.at[p], sems[p])
        return hl, hy

    def _hist_group(p, i):
        lg = lbuf[p, pl.ds(i * L, L)]
        yi = ybuf[p, pl.ds(i * L, L)]
        yf = yi.astype(jnp.float32)
        t = lg * yf
        e = (1.0 + lg) - 2.0 * t
        s = e * SCALE + 1.0
        s = jnp.minimum(s, float(B))
        s = jnp.maximum(s, 0.0)
        v = s.astype(jnp.int32)
        idx = (2 * B) - (v + v) + yi
        plsc.addupdate_scatter(cnt, [idx], ones)

    pend = _start(0)
    for ch in range(NCHUNKS):
        nxt = _start(ch + 1) if ch + 1 < NCHUNKS else None
        pend[0].wait()
        pend[1].wait()
        plsc.parallel_loop(0, CHUNK // L, unroll=U)(
            functools.partial(_hist_group, ch & 1))
        pend = nxt

    oddmask = (iota & 1).astype(jnp.float32)

    def _gsum(i, acc):
        return acc + cnt[pl.ds(i * L, L)] * oddmask

    g = jnp.sum(plsc.parallel_loop(0, CNT_TOT // L, unroll=4, carry=zeros)(_gsum))
    g2 = float(N) - g

    def _final(b, carry):
        pcar, ncar, acc = carry
        base = 2 * (b * L) + 2 * iota
        cn = plsc.load_gather(cnt, [base])
        cp = plsc.load_gather(cnt, [base + 1])
        pi = plsc.cumsum(cp) + pcar
        ni = plsc.cumsum(cn) + ncar
        pe = pi - cp
        ne = ni - cn
        den1 = jnp.maximum((g + ne) * (g + ne + cn), 1.0)
        mass1 = ((g - pe) * cn + cp * (g + ne)) / den1
        den2 = jnp.maximum((g2 + pe) * (g2 + pe + cp), 1.0)
        mass2 = ((g2 - ne) * cp + cn * (g2 + pe)) / den2
        bg = (b * L + iota).astype(jnp.float32)
        center = (B - 0.5) / SCALE - bg * (1.0 / SCALE)
        acc = acc + center * (mass1 + mass2)
        return pcar + jnp.sum(cp), ncar + jnp.sum(cn), acc

    _, _, acc = lax.fori_loop(
        0, B // L, _final, (jnp.float32(0.0), jnp.float32(0.0), zeros)
    )

    partial = jnp.sum(acc) * active * (1.0 / (2.0 * C))
    ostage[...] = jnp.where(iota == 0, partial, 0.0)
    pltpu.sync_copy(ostage, out.at[wid])


def kernel(logits, labels):
    logits_t = logits.T
    labels_t = labels.astype(jnp.int32).T
    parts = _lovasz_sc(logits_t, labels_t)
    return jnp.sum(parts)
